# Initial kernel scaffold; baseline (speedup 1.0000x reference)
import jax
import jax.numpy as jnp
from jax.experimental import pallas as pl


def kernel(x, edge_index, edge_attr, W, b):
    raise NotImplementedError("write your pallas kernel here")


# speedup vs baseline: 4.7198x; 4.7198x over previous
import functools

import jax
import jax.numpy as jnp
from jax import lax
from jax.experimental import pallas as pl
from jax.experimental.pallas import tpu as pltpu
from jax.experimental.pallas import tpu_sc as plsc

N = 10000
E = 320000
D = 128
NC = 2
NS = 16
NW = NC * NS
EPW = E // NW
CH = 80
NCH = EPW // CH
IB = 5
NIB = NCH // IB
RPT = 624
TAIL = N - NS * RPT


def _matmul_body(x_ref, w_ref, o_ref):
    o_ref[...] = lax.dot_general(
        x_ref[...], w_ref[...], (((1,), (1,)), ((), ())),
        preferred_element_type=jnp.float32)


def _project(x, W):
    return pl.pallas_call(
        _matmul_body,
        grid=(10,),
        in_specs=[
            pl.BlockSpec((N // 10, D), lambda i: (i, 0)),
            pl.BlockSpec((D, D), lambda i: (0, 0)),
        ],
        out_specs=pl.BlockSpec((N // 10, D), lambda i: (i, 0)),
        out_shape=jax.ShapeDtypeStruct((N, D), jnp.float32),
    )(x, W)


_mesh = plsc.VectorSubcoreMesh(
    core_axis_name="c", subcore_axis_name="s", num_cores=NC, num_subcores=NS)


@functools.partial(
    pl.kernel,
    out_type=jax.ShapeDtypeStruct((NC, N, D), jnp.float32),
    mesh=_mesh,
    scratch_types=[
        pltpu.VMEM((IB, CH), jnp.int32),
        pltpu.VMEM((IB, CH), jnp.int32),
        pltpu.VMEM((CH, D), jnp.float32),
        pltpu.VMEM((CH, D), jnp.float32),
        pltpu.VMEM_SHARED((N, D), jnp.float32),
        pltpu.SemaphoreType.DMA,
    ],
)
def _message_pass(xw_hbm, src_hbm, dst_hbm, ea_hbm, out_hbm,
                  src_v, dst_v, xj_v, ea_v, acc, sem):
    c = lax.axis_index("c")
    s = lax.axis_index("s")
    wid = s * NC + c
    ebase = wid * EPW

    zero16 = jnp.zeros((16,), jnp.float32)

    def zero_body(r, zcarry):
        for k in range(D // 16):
            ea_v[r, pl.ds(k * 16, 16)] = zero16
        return zcarry

    lax.fori_loop(0, CH, zero_body, 0)
    for i in range(RPT // CH):
        pltpu.sync_copy(ea_v, acc.at[pl.ds(s * RPT + i * CH, CH)])
    rem = RPT - (RPT // CH) * CH
    pltpu.sync_copy(ea_v.at[pl.ds(0, rem)],
                    acc.at[pl.ds(s * RPT + RPT - rem, rem)])

    @pl.when(s == 0)
    def _():
        pltpu.sync_copy(ea_v.at[pl.ds(0, TAIL)],
                        acc.at[pl.ds(NS * RPT, TAIL)])

    plsc.subcore_barrier()

    def blk_body(bi, bcarry):
        pltpu.sync_copy(src_hbm.at[wid, bi], src_v)
        pltpu.sync_copy(dst_hbm.at[wid, bi], dst_v)

        def chunk_body(jj, carry):
            j = bi * IB + jj
            gat = pltpu.async_copy(xw_hbm.at[src_v.at[jj]], xj_v, sem)
            pltpu.sync_copy(ea_hbm.at[pl.ds(ebase + j * CH, CH)], ea_v)
            gat.wait()

            def row_body(r, rcarry):
                for k in range(D // 16):
                    sl = pl.ds(k * 16, 16)
                    xj_v[r, sl] = jnp.maximum(xj_v[r, sl] + ea_v[r, sl],
                                              0.0)
                return rcarry

            lax.fori_loop(0, CH, row_body, 0)
            pltpu.sync_copy(xj_v, acc.at[dst_v.at[jj]], add=True)
            return carry

        lax.fori_loop(0, IB, chunk_body, 0)
        return bcarry

    lax.fori_loop(0, NIB, blk_body, 0)
    plsc.subcore_barrier()

    pltpu.sync_copy(acc.at[pl.ds(s * RPT, RPT)],
                    out_hbm.at[c, pl.ds(s * RPT, RPT)])

    @pl.when(s == 0)
    def _():
        pltpu.sync_copy(acc.at[pl.ds(NS * RPT, TAIL)],
                        out_hbm.at[c, pl.ds(NS * RPT, TAIL)])


def _combine_body(p_ref, b_ref, o_ref):
    o_ref[...] = p_ref[0] + p_ref[1] + b_ref[...]


def _combine(partials, b2d):
    return pl.pallas_call(
        _combine_body,
        grid=(10,),
        in_specs=[
            pl.BlockSpec((NC, N // 10, D), lambda i: (0, i, 0)),
            pl.BlockSpec((1, D), lambda i: (0, 0)),
        ],
        out_specs=pl.BlockSpec((N // 10, D), lambda i: (i, 0)),
        out_shape=jax.ShapeDtypeStruct((N, D), jnp.float32),
    )(partials, b2d)


def kernel(x, edge_index, edge_attr, W, b):
    src = edge_index[0].reshape(NW, NIB, IB, CH)
    dst = edge_index[1].reshape(NW, NIB, IB, CH)
    xw = _project(x, W)
    partials = _message_pass(xw, src, dst, edge_attr)
    return _combine(partials, b.reshape(1, D))
